# baseline (speedup 1.0000x reference)
import functools

import jax
import jax.numpy as jnp
from jax import lax
from jax.experimental import pallas as pl
from jax.experimental.pallas import tpu as pltpu

_HEAD_DIM = 128


def _mha_kernel(x_ref, w_ref, b_ref, o_ref, qkv_ref, va_ref, *, n_heads, e):
    d = _HEAD_DIM
    x = x_ref[0].astype(jnp.bfloat16)
    qkv_ref[...] = (jnp.dot(x, w_ref[...], preferred_element_type=jnp.float32)
                    + b_ref[...]).astype(jnp.bfloat16)
    va_ref[:, d:] = jnp.ones_like(va_ref[:, d:])

    for h in range(n_heads):
        q = qkv_ref[:, h * d:(h + 1) * d]
        k = qkv_ref[:, e + h * d:e + (h + 1) * d]
        va_ref[:, :d] = qkv_ref[:, 2 * e + h * d:2 * e + (h + 1) * d]
        s = lax.dot_general(q, k, (((1,), (1,)), ((), ())),
                            preferred_element_type=jnp.float32)
        m = jnp.max(s, axis=-1, keepdims=True)
        p = jnp.exp(s - m).astype(jnp.bfloat16)
        r = jnp.dot(p, va_ref[...], preferred_element_type=jnp.float32)
        o_ref[0, :, h * d:(h + 1) * d] = r[:, :d] / r[:, d:d + 1]


def kernel(x, w_qkv, b_qkv):
    B, S, E = x.shape
    n_heads = E // _HEAD_DIM
    wb = w_qkv.astype(jnp.bfloat16)
    return pl.pallas_call(
        functools.partial(_mha_kernel, n_heads=n_heads, e=E),
        out_shape=jax.ShapeDtypeStruct((B, S, E), x.dtype),
        grid=(B,),
        in_specs=[
            pl.BlockSpec((1, S, E), lambda b: (b, 0, 0)),
            pl.BlockSpec((E, 3 * E), lambda b: (0, 0)),
            pl.BlockSpec((1, 3 * E), lambda b: (0, 0)),
        ],
        out_specs=pl.BlockSpec((1, S, E), lambda b: (b, 0, 0)),
        scratch_shapes=[
            pltpu.VMEM((S, 3 * E), jnp.bfloat16),
            pltpu.VMEM((S, 2 * _HEAD_DIM), jnp.bfloat16),
        ],
        compiler_params=pltpu.CompilerParams(
            dimension_semantics=("parallel",)),
    )(x, wb, b_qkv)


# speedup vs baseline: 1.1627x; 1.1627x over previous
import functools

import jax
import jax.numpy as jnp
from jax import lax
from jax.experimental import pallas as pl
from jax.experimental.pallas import tpu as pltpu

_HEAD_DIM = 128


def _mha_kernel(x_ref, w_ref, b_ref, o_ref, qkv_ref, wb_ref, *, n_heads, e):
    d = _HEAD_DIM

    @pl.when(pl.program_id(0) == 0)
    def _stage_weights():
        wb_ref[...] = w_ref[...].astype(jnp.bfloat16)

    x = x_ref[0].astype(jnp.bfloat16)
    qkv_ref[...] = jnp.dot(
        x, wb_ref[...], preferred_element_type=jnp.float32) + b_ref[...]

    for h in range(n_heads):
        q = qkv_ref[:, h * d:(h + 1) * d].astype(jnp.bfloat16)
        k = qkv_ref[:, e + h * d:e + (h + 1) * d].astype(jnp.bfloat16)
        v = qkv_ref[:, 2 * e + h * d:2 * e + (h + 1) * d].astype(jnp.bfloat16)
        s = lax.dot_general(q, k, (((1,), (1,)), ((), ())),
                            preferred_element_type=jnp.float32)
        m = jnp.max(s, axis=-1, keepdims=True)
        p = jnp.exp(s - m)
        acc = jnp.dot(p.astype(jnp.bfloat16), v,
                      preferred_element_type=jnp.float32)
        l = jnp.sum(p, axis=-1, keepdims=True)
        o_ref[0, :, h * d:(h + 1) * d] = (acc / l).astype(o_ref.dtype)


def kernel(x, w_qkv, b_qkv):
    B, S, E = x.shape
    n_heads = E // _HEAD_DIM
    return pl.pallas_call(
        functools.partial(_mha_kernel, n_heads=n_heads, e=E),
        out_shape=jax.ShapeDtypeStruct((B, S, E), x.dtype),
        grid=(B,),
        in_specs=[
            pl.BlockSpec((1, S, E), lambda b: (b, 0, 0)),
            pl.BlockSpec((E, 3 * E), lambda b: (0, 0)),
            pl.BlockSpec((1, 3 * E), lambda b: (0, 0)),
        ],
        out_specs=pl.BlockSpec((1, S, E), lambda b: (b, 0, 0)),
        scratch_shapes=[
            pltpu.VMEM((S, 3 * E), jnp.float32),
            pltpu.VMEM((E, 3 * E), jnp.bfloat16),
        ],
        compiler_params=pltpu.CompilerParams(
            dimension_semantics=("arbitrary",)),
    )(x, w_qkv, b_qkv)
